# baseline (speedup 1.0000x reference)
import dataclasses
import functools

import jax
import jax.numpy as jnp
import numpy as np
from jax import lax
from jax.experimental import pallas as pl
from jax.experimental.pallas import tpu as pltpu
from jax.experimental.pallas import tpu_sc as plsc

_D = 128
_SEQ = 50
_B = 4096
_OUT = 512
_NC = 2
_NS = 16
_NW = _NC * _NS
_ROWS_PER_TILE = _B // _NW
_ROWS_PER_GATHER = 2
_IDX_PER_GATHER = _ROWS_PER_GATHER * _SEQ
_GATHERS_PER_TILE = _ROWS_PER_TILE // _ROWS_PER_GATHER
_LANES = 16
_NBUF = 2

_PERM = np.concatenate(
    [np.concatenate([g * 32 + np.arange(0, 32, 2),
                     g * 32 + np.arange(1, 32, 2)])
     for g in range(_D // 32)])


def _sc_pool(table_bf16, idx_flat):
    mesh = plsc.VectorSubcoreMesh(core_axis_name="c", subcore_axis_name="s")
    cp = pltpu.CompilerParams()
    if "needs_layout_passes" in pltpu.CompilerParams.__dataclass_fields__:
        cp = dataclasses.replace(cp, needs_layout_passes=False)
    if "use_tc_tiling_on_sc" in pltpu.CompilerParams.__dataclass_fields__:
        cp = dataclasses.replace(cp, use_tc_tiling_on_sc=False)

    @functools.partial(
        pl.kernel,
        mesh=mesh,
        compiler_params=cp,
        out_type=jax.ShapeDtypeStruct((_B, _D), jnp.float32),
        scratch_types=(
            [pltpu.VMEM((_GATHERS_PER_TILE, _IDX_PER_GATHER), jnp.int32)]
            + [pltpu.VMEM((_IDX_PER_GATHER, _D), jnp.float32)] * _NBUF
            + [pltpu.VMEM((_ROWS_PER_TILE, _D), jnp.float32)]
            + [pltpu.SemaphoreType.DMA] * _NBUF
        ),
    )
    def k(table_hbm, idx_hbm, out_hbm, idx_v, *rest):
        bufs = rest[:_NBUF]
        out_v = rest[_NBUF]
        sems = rest[_NBUF + 1:]
        wid = lax.axis_index("s") * _NC + lax.axis_index("c")
        row0 = wid * _ROWS_PER_TILE
        pltpu.sync_copy(
            idx_hbm.at[pl.ds(wid * _GATHERS_PER_TILE, _GATHERS_PER_TILE)],
            idx_v)

        def src(g):
            return table_hbm.at[idx_v.at[g]]

        def fire(g, b):
            pltpu.async_copy(src(g), bufs[b], sems[b])

        def wait(g, b):
            pltpu.make_async_copy(src(g), bufs[b], sems[b]).wait()

        def reduce(buf, g):
            @pl.loop(0, _D // _LANES)
            def _reduce(cc):
                c0 = cc * _LANES
                for r in range(_ROWS_PER_GATHER):
                    base = r * _SEQ
                    accs = [buf[base + a, pl.ds(c0, _LANES)]
                            for a in range(4)]
                    for l in range(4, _SEQ):
                        accs[l % 4] = accs[l % 4] + buf[base + l,
                                                        pl.ds(c0, _LANES)]
                    out_v[g * _ROWS_PER_GATHER + r, pl.ds(c0, _LANES)] = (
                        ((accs[0] + accs[1]) + (accs[2] + accs[3]))
                        * (1.0 / _SEQ))

        for b in range(_NBUF):
            fire(b, b)

        @pl.loop(0, _GATHERS_PER_TILE - _NBUF, step=_NBUF)
        def _main(g0):
            for b in range(_NBUF):
                g = g0 + b
                wait(g, b)
                reduce(bufs[b], g)
                fire(g + _NBUF, b)

        for b in range(_NBUF):
            g = _GATHERS_PER_TILE - _NBUF + b
            wait(g, b)
            reduce(bufs[b], g)

        pltpu.sync_copy(out_v, out_hbm.at[pl.ds(row0, _ROWS_PER_TILE)])

    return k(table_bf16, idx_flat)


def _tc_project(pooled, w, bias):
    blk = 2048

    def body(p_ref, w_ref, b_ref, o_ref):
        o_ref[...] = lax.dot_general(
            p_ref[...], w_ref[...], (((1,), (0,)), ((), ())),
            preferred_element_type=jnp.float32,
            precision=lax.Precision.DEFAULT) + b_ref[...]

    return pl.pallas_call(
        body,
        grid=(_B // blk,),
        in_specs=[
            pl.BlockSpec((blk, _D), lambda i: (i, 0)),
            pl.BlockSpec((_D, _OUT), lambda i: (0, 0)),
            pl.BlockSpec((1, _OUT), lambda i: (0, 0)),
        ],
        out_specs=pl.BlockSpec((blk, _OUT), lambda i: (i, 0)),
        out_shape=jax.ShapeDtypeStruct((_B, _OUT), jnp.float32),
    )(pooled, w, bias.reshape(1, _OUT))


def kernel(x, table, W, b):
    xp = x.astype(jnp.int32).reshape(_B // _ROWS_PER_GATHER, _IDX_PER_GATHER)
    pooled = _sc_pool(table, xp)
    return _tc_project(pooled, W, b)


# speedup vs baseline: 1.2752x; 1.2752x over previous
import dataclasses
import functools

import jax
import jax.numpy as jnp
import numpy as np
from jax import lax
from jax.experimental import pallas as pl
from jax.experimental.pallas import tpu as pltpu
from jax.experimental.pallas import tpu_sc as plsc

_D = 128
_SEQ = 50
_B = 4096
_OUT = 512
_NC = 2
_NS = 16
_NW = _NC * _NS
_ROWS_PER_TILE = _B // _NW
_ROWS_PER_GATHER = 2
_IDX_PER_GATHER = _ROWS_PER_GATHER * _SEQ
_GATHERS_PER_TILE = _ROWS_PER_TILE // _ROWS_PER_GATHER
_LANES = 16
_NBUF = 4

_PERM = np.concatenate(
    [np.concatenate([g * 32 + np.arange(0, 32, 2),
                     g * 32 + np.arange(1, 32, 2)])
     for g in range(_D // 32)])


def _sc_pool(table_bf16, idx_flat):
    mesh = plsc.VectorSubcoreMesh(core_axis_name="c", subcore_axis_name="s")
    cp = pltpu.CompilerParams()
    if "needs_layout_passes" in pltpu.CompilerParams.__dataclass_fields__:
        cp = dataclasses.replace(cp, needs_layout_passes=False)
    if "use_tc_tiling_on_sc" in pltpu.CompilerParams.__dataclass_fields__:
        cp = dataclasses.replace(cp, use_tc_tiling_on_sc=False)

    @functools.partial(
        pl.kernel,
        mesh=mesh,
        compiler_params=cp,
        out_type=jax.ShapeDtypeStruct((_B, _D), jnp.float32),
        scratch_types=(
            [pltpu.VMEM((_GATHERS_PER_TILE, _IDX_PER_GATHER), jnp.int32)]
            + [pltpu.VMEM((_IDX_PER_GATHER, _D), jnp.float32)] * _NBUF
            + [pltpu.VMEM((_ROWS_PER_TILE, _D), jnp.float32)]
            + [pltpu.SemaphoreType.DMA] * _NBUF
        ),
    )
    def k(table_hbm, idx_hbm, out_hbm, idx_v, *rest):
        bufs = rest[:_NBUF]
        out_v = rest[_NBUF]
        sems = rest[_NBUF + 1:]
        wid = lax.axis_index("s") * _NC + lax.axis_index("c")
        row0 = wid * _ROWS_PER_TILE
        pltpu.sync_copy(
            idx_hbm.at[pl.ds(wid * _GATHERS_PER_TILE, _GATHERS_PER_TILE)],
            idx_v)

        def src(g):
            return table_hbm.at[idx_v.at[g]]

        def fire(g, b):
            pltpu.async_copy(src(g), bufs[b], sems[b])

        def wait(g, b):
            pltpu.make_async_copy(src(g), bufs[b], sems[b]).wait()

        def reduce(buf, g):
            @pl.loop(0, _D // _LANES)
            def _reduce(cc):
                c0 = cc * _LANES
                for r in range(_ROWS_PER_GATHER):
                    base = r * _SEQ
                    accs = [buf[base + a, pl.ds(c0, _LANES)]
                            for a in range(4)]
                    for l in range(4, _SEQ):
                        accs[l % 4] = accs[l % 4] + buf[base + l,
                                                        pl.ds(c0, _LANES)]
                    out_v[g * _ROWS_PER_GATHER + r, pl.ds(c0, _LANES)] = (
                        ((accs[0] + accs[1]) + (accs[2] + accs[3]))
                        * (1.0 / _SEQ))

        for b in range(_NBUF):
            fire(b, b)

        @pl.loop(0, _GATHERS_PER_TILE - _NBUF, step=_NBUF)
        def _main(g0):
            for b in range(_NBUF):
                g = g0 + b
                wait(g, b)
                reduce(bufs[b], g)
                fire(g + _NBUF, b)

        for b in range(_NBUF):
            g = _GATHERS_PER_TILE - _NBUF + b
            wait(g, b)
            reduce(bufs[b], g)

        pltpu.sync_copy(out_v, out_hbm.at[pl.ds(row0, _ROWS_PER_TILE)])

    return k(table_bf16, idx_flat)


def _tc_project(pooled, w, bias):
    blk = 2048

    def body(p_ref, w_ref, b_ref, o_ref):
        o_ref[...] = lax.dot_general(
            p_ref[...], w_ref[...], (((1,), (0,)), ((), ())),
            preferred_element_type=jnp.float32,
            precision=lax.Precision.DEFAULT) + b_ref[...]

    return pl.pallas_call(
        body,
        grid=(_B // blk,),
        in_specs=[
            pl.BlockSpec((blk, _D), lambda i: (i, 0)),
            pl.BlockSpec((_D, _OUT), lambda i: (0, 0)),
            pl.BlockSpec((1, _OUT), lambda i: (0, 0)),
        ],
        out_specs=pl.BlockSpec((blk, _OUT), lambda i: (i, 0)),
        out_shape=jax.ShapeDtypeStruct((_B, _OUT), jnp.float32),
    )(pooled, w, bias.reshape(1, _OUT))


def kernel(x, table, W, b):
    xp = x.astype(jnp.int32).reshape(_B // _ROWS_PER_GATHER, _IDX_PER_GATHER)
    pooled = _sc_pool(table, xp)
    return _tc_project(pooled, W, b)
